# baseline (speedup 1.0000x reference)
import jax
import jax.numpy as jnp
from jax import lax
from jax.experimental import pallas as pl
from jax.experimental.pallas import tpu as pltpu
from jax.experimental.pallas import tpu_sc as plsc

_CH = 16
_BS_SC = 8
_CPW = _BS_SC * 16 // 32


def _qpool_sc_body(rho_hbm, out_hbm,
                   rbuf0, rbuf1, cdbuf0, cdbuf1, obuf0, obuf1,
                   semr0, semr1, semc0, semc1):
    cid = lax.axis_index("c")
    sid = lax.axis_index("s")
    wid = sid * 2 + cid
    g0 = wid * _CPW
    lanes = lax.iota(jnp.int32, 16)

    ins = [(rbuf0, cdbuf0, semr0, semc0), (rbuf1, cdbuf1, semr1, semc1)]
    obufs = [obuf0, obuf1]

    def issue(g, p):
        rbuf, cdbuf, semr, semc = ins[p]
        bat = g >> 4
        r0 = 64 * (g & 15)
        pltpu.async_copy(rho_hbm.at[bat, pl.ds(r0, 32)], rbuf, semr)
        pltpu.async_copy(
            rho_hbm.at[bat, pl.ds(r0 + 32, 32),
                       pl.ds((r0 + 32) // 128 * 128, 128)],
            cdbuf, semc)

    def wait_in(p):
        rbuf, cdbuf, semr, semc = ins[p]
        pltpu.make_async_copy(rho_hbm.at[0, pl.ds(0, 32)],
                              rbuf, semr).wait()
        pltpu.make_async_copy(rho_hbm.at[0, pl.ds(0, 32), pl.ds(0, 128)],
                              cdbuf, semc).wait()

    def compute(g, p):
        rbuf, cdbuf, _, _ = ins[p]
        obuf = obufs[p]
        bat = g >> 4
        i0 = g & 15
        base16 = 16 * i0
        off = (64 * i0 + 32) % 128

        def row_body(t, carry2):
            tf = jnp.full((16,), t, jnp.int32)
            te = 2 * tf
            to = te + 1

            for lp in range(16):
                av = plsc.load_gather(rbuf, [te, 64 * lp + 2 * lanes])
                obuf[t, pl.ds(16 * lp, 16)] = av

            bv = plsc.load_gather(rbuf, [to, 64 * lanes + 2 * t + 1])
            plsc.addupdate_scatter(obuf, [tf, 16 * lanes + t], bv)

            cv = plsc.load_gather(cdbuf, [te, off + 2 * lanes])
            dv = plsc.load_gather(cdbuf, [to, jnp.full((16,), off,
                                                       jnp.int32) + 2 * t + 1])
            cd = cv + jnp.where(lanes == t, dv, jnp.float32(0))
            plsc.addupdate_scatter(obuf, [tf, base16 + lanes], cd)
            return carry2
        lax.fori_loop(0, _CH, row_body, 0)

        pltpu.sync_copy(obuf, out_hbm.at[bat, pl.ds(base16, _CH)])

    issue(g0, 0)
    issue(g0 + 1, 1)

    def pair_body(kk, carry):
        for p in range(2):
            g = g0 + 2 * kk + p
            wait_in(p)
            compute(g, p)

            @pl.when(kk < (_CPW // 2) - 1)
            def _():
                issue(g + 2, p)
        return carry
    lax.fori_loop(0, _CPW // 2, pair_body, 0)


def _qpool_sc(rho):
    return pl.kernel(
        _qpool_sc_body,
        out_type=jax.ShapeDtypeStruct((_BS_SC, 256, 256), jnp.float32),
        mesh=plsc.VectorSubcoreMesh(core_axis_name="c", subcore_axis_name="s"),
        scratch_types=(
            [pltpu.VMEM((32, 1024), jnp.float32)] * 2
            + [pltpu.VMEM((32, 128), jnp.float32)] * 2
            + [pltpu.VMEM((_CH, 256), jnp.float32)] * 2
            + [pltpu.SemaphoreType.DMA] * 4
        ),
        compiler_params=pltpu.CompilerParams(use_tc_tiling_on_sc=True,
                                             needs_layout_passes=False),
    )(rho)


def _iota2(shape, dim):
    return lax.broadcasted_iota(jnp.int32, shape, dim)


def _build_selectors():
    import numpy as np
    y = np.arange(256)
    gy = y // 16
    my = y % 16
    c = np.arange(1024)[:, None]
    sel_a = (c == (64 * gy + 2 * my)[None, :]).astype(np.float32)
    sel_b = (c == (64 * gy + 2 * my + 1)[None, :]).astype(np.float32)
    sel_c = (c == (64 * gy + 32 + 2 * my)[None, :]).astype(np.float32)
    sel_d = (c == (64 * gy + 33 + 2 * my)[None, :]).astype(np.float32)
    x = np.arange(256)
    rx = (64 * (x // 16) + 2 * (x % 16))[:, None]
    r = np.arange(1024)[None, :]
    rs_e = (r == rx).astype(np.float32)
    rs_o = (r == rx + 1).astype(np.float32)
    rs_ce = (r == rx + 32).astype(np.float32)
    rs_co = (r == rx + 33).astype(np.float32)
    sel = np.concatenate([sel_a, sel_b, sel_c, sel_d], axis=1)
    rs = np.concatenate([rs_e, rs_o, rs_ce, rs_co], axis=0)
    xg, yg = x[:, None] // 16, x[None, :] // 16
    xm, ym = x[:, None] % 16, x[None, :] % 16
    bm = (ym == xm).astype(np.float32)
    gm = (yg == xg).astype(np.float32)
    masks = np.stack([bm, gm, bm * gm])
    return jnp.asarray(sel), jnp.asarray(rs), jnp.asarray(masks)


def _qpool_tc_body(blk_ref, sel_ref, rs_ref, masks_ref, o_ref):
    f32 = jnp.float32
    blk = blk_ref[0]

    g = jnp.dot(blk, sel_ref[...], preferred_element_type=f32)

    a_f = jnp.dot(rs_ref[0:0+256, :], g[:, 0:0+256],
                  preferred_element_type=f32)
    o_f = jnp.dot(rs_ref[256:256+256, :], g[:, 256:256+256],
                  preferred_element_type=f32)
    c_f = jnp.dot(rs_ref[512:512+256, :], g[:, 512:512+256],
                  preferred_element_type=f32)
    d_f = jnp.dot(rs_ref[768:768+256, :], g[:, 768:768+256],
                  preferred_element_type=f32)

    xmod = _iota2((256, 256), 0) % 16
    xgrp = _iota2((256, 256), 0) // 16
    ymod = _iota2((256, 256), 1) % 16
    ygrp = _iota2((256, 256), 1) // 16
    bmask = ymod == xmod
    gmask = ygrp == xgrp
    zero = jnp.float32(0)
    out = (a_f + jnp.where(bmask, o_f, zero)
           + jnp.where(gmask, c_f, zero)
           + jnp.where(bmask & gmask, d_f, zero))
    o_ref[0] = out


def _qpool_tc(rho):
    b = rho.shape[0]
    nb = b - _BS_SC
    sel, rs, masks = _build_selectors()
    return pl.pallas_call(
        _qpool_tc_body,
        grid=(nb,),
        in_specs=[
            pl.BlockSpec((1, 1024, 1024), lambda bb: (bb + _BS_SC, 0, 0)),
            pl.BlockSpec((1024, 1024), lambda bb: (0, 0)),
            pl.BlockSpec((1024, 1024), lambda bb: (0, 0)),
            pl.BlockSpec((3, 256, 256), lambda bb: (0, 0, 0)),
        ],
        out_specs=pl.BlockSpec((1, 256, 256), lambda bb: (bb, 0, 0)),
        out_shape=jax.ShapeDtypeStruct((nb, 256, 256), jnp.float32),
    )(rho, sel, rs, masks)


def kernel(rho, mask_x, mask_y, new_x, new_y):
    out_sc = _qpool_sc(rho)
    out_tc = _qpool_tc(rho)
    return jnp.concatenate([out_sc, out_tc], axis=0)


# speedup vs baseline: 1.3127x; 1.3127x over previous
import jax
import jax.numpy as jnp
from jax import lax
from jax.experimental import pallas as pl
from jax.experimental.pallas import tpu as pltpu
from jax.experimental.pallas import tpu_sc as plsc

_CH = 16
_BS_SC = 16
_CPW = _BS_SC * 16 // 32


def _qpool_sc_body(rho_hbm, out_hbm,
                   rbuf0, rbuf1, cdbuf0, cdbuf1, obuf0, obuf1,
                   semr0, semr1, semc0, semc1):
    cid = lax.axis_index("c")
    sid = lax.axis_index("s")
    wid = sid * 2 + cid
    g0 = wid * _CPW
    lanes = lax.iota(jnp.int32, 16)

    ins = [(rbuf0, cdbuf0, semr0, semc0), (rbuf1, cdbuf1, semr1, semc1)]
    obufs = [obuf0, obuf1]

    def issue(g, p):
        rbuf, cdbuf, semr, semc = ins[p]
        bat = g >> 4
        r0 = 64 * (g & 15)
        pltpu.async_copy(rho_hbm.at[bat, pl.ds(r0, 32)], rbuf, semr)
        pltpu.async_copy(
            rho_hbm.at[bat, pl.ds(r0 + 32, 32),
                       pl.ds((r0 + 32) // 128 * 128, 128)],
            cdbuf, semc)

    def wait_in(p):
        rbuf, cdbuf, semr, semc = ins[p]
        pltpu.make_async_copy(rho_hbm.at[0, pl.ds(0, 32)],
                              rbuf, semr).wait()
        pltpu.make_async_copy(rho_hbm.at[0, pl.ds(0, 32), pl.ds(0, 128)],
                              cdbuf, semc).wait()

    def compute(g, p):
        rbuf, cdbuf, _, _ = ins[p]
        obuf = obufs[p]
        bat = g >> 4
        i0 = g & 15
        base16 = 16 * i0
        off = (64 * i0 + 32) % 128

        def row_body(t, carry2):
            tf = jnp.full((16,), t, jnp.int32)
            te = 2 * tf
            to = te + 1

            for lp in range(16):
                av = plsc.load_gather(rbuf, [te, 64 * lp + 2 * lanes])
                obuf[t, pl.ds(16 * lp, 16)] = av

            bv = plsc.load_gather(rbuf, [to, 64 * lanes + 2 * t + 1])
            plsc.addupdate_scatter(obuf, [tf, 16 * lanes + t], bv)

            cv = plsc.load_gather(cdbuf, [te, off + 2 * lanes])
            dv = plsc.load_gather(cdbuf, [to, jnp.full((16,), off,
                                                       jnp.int32) + 2 * t + 1])
            cd = cv + jnp.where(lanes == t, dv, jnp.float32(0))
            plsc.addupdate_scatter(obuf, [tf, base16 + lanes], cd)
            return carry2
        lax.fori_loop(0, _CH, row_body, 0)

        pltpu.sync_copy(obuf, out_hbm.at[bat, pl.ds(base16, _CH)])

    issue(g0, 0)
    issue(g0 + 1, 1)

    def pair_body(kk, carry):
        for p in range(2):
            g = g0 + 2 * kk + p
            wait_in(p)
            compute(g, p)

            @pl.when(kk < (_CPW // 2) - 1)
            def _():
                issue(g + 2, p)
        return carry
    lax.fori_loop(0, _CPW // 2, pair_body, 0)


def _qpool_sc(rho):
    return pl.kernel(
        _qpool_sc_body,
        out_type=jax.ShapeDtypeStruct((_BS_SC, 256, 256), jnp.float32),
        mesh=plsc.VectorSubcoreMesh(core_axis_name="c", subcore_axis_name="s"),
        scratch_types=(
            [pltpu.VMEM((32, 1024), jnp.float32)] * 2
            + [pltpu.VMEM((32, 128), jnp.float32)] * 2
            + [pltpu.VMEM((_CH, 256), jnp.float32)] * 2
            + [pltpu.SemaphoreType.DMA] * 4
        ),
        compiler_params=pltpu.CompilerParams(use_tc_tiling_on_sc=True,
                                             needs_layout_passes=False),
    )(rho)


def kernel(rho, mask_x, mask_y, new_x, new_y):
    return _qpool_sc(rho)
